# baseline (speedup 1.0000x reference)
import functools

import jax
import jax.numpy as jnp
from jax import lax
from jax.experimental import pallas as pl
from jax.experimental.pallas import tpu as pltpu
from jax.experimental.pallas import tpu_sc as plsc

_DIM = 32
_NX = 4096
_NY = 4096

_NC = 2
_NS = 16
_NW = _NC * _NS
_BPW = _NX // _NW


def _gather_body(E_hbm, xidx_hbm, yidx_hbm, ex_hbm, ey_hbm,
                 xi_v, xrows_v, yi_v, yrows_v, semx, semy):
    wid = lax.axis_index("s") * _NC + lax.axis_index("c")
    base = wid * _BPW
    pltpu.sync_copy(xidx_hbm.at[pl.ds(base, _BPW)], xi_v)
    pltpu.sync_copy(yidx_hbm.at[pl.ds(base, _BPW)], yi_v)
    cx = pltpu.async_copy(E_hbm.at[xi_v], xrows_v, semx)
    cy = pltpu.async_copy(E_hbm.at[yi_v], yrows_v, semy)
    cx.wait()
    cy.wait()
    pltpu.sync_copy(xrows_v, ex_hbm.at[pl.ds(base, _BPW)])
    pltpu.sync_copy(yrows_v, ey_hbm.at[pl.ds(base, _BPW)])


@functools.cache
def _make_gather():
    return functools.partial(
        pl.kernel,
        mesh=plsc.VectorSubcoreMesh(core_axis_name="c", subcore_axis_name="s"),
        compiler_params=pltpu.CompilerParams(use_tc_tiling_on_sc=False),
        out_type=[
            jax.ShapeDtypeStruct((_NX, _DIM), jnp.float32),
            jax.ShapeDtypeStruct((_NY, _DIM), jnp.float32),
        ],
        scratch_types=[
            pltpu.VMEM((_BPW,), jnp.int32),
            pltpu.VMEM((_BPW, _DIM), jnp.float32),
            pltpu.VMEM((_BPW,), jnp.int32),
            pltpu.VMEM((_BPW, _DIM), jnp.float32),
            pltpu.SemaphoreType.DMA,
            pltpu.SemaphoreType.DMA,
        ],
    )(_gather_body)


_BX = 512


def _sim_body(ex_ref, ey_ref, out_ref):
    ex = ex_ref[...]
    ey = ey_ref[...]
    exn = ex / (jnp.sqrt(jnp.sum(ex * ex, axis=1, keepdims=True)) + 1e-8)
    eyn = ey / (jnp.sqrt(jnp.sum(ey * ey, axis=1, keepdims=True)) + 1e-8)
    out_ref[...] = lax.dot_general(
        exn, eyn, (((1,), (1,)), ((), ())),
        preferred_element_type=jnp.float32)


_sim = pl.pallas_call(
    _sim_body,
    grid=(_NX // _BX,),
    in_specs=[
        pl.BlockSpec((_BX, _DIM), lambda i: (i, 0)),
        pl.BlockSpec((_NY, _DIM), lambda i: (0, 0)),
    ],
    out_specs=pl.BlockSpec((_BX, _NY), lambda i: (i, 0)),
    out_shape=jax.ShapeDtypeStruct((_NX, _NY), jnp.float32),
)


def kernel(x_idx, y_idx, E):
    ex, ey = _make_gather()(E, x_idx.astype(jnp.int32), y_idx.astype(jnp.int32))
    return _sim(ex, ey)
